# baseline (speedup 1.0000x reference)
import functools

import jax
import jax.numpy as jnp
from jax import lax
from jax.experimental import pallas as pl
from jax.experimental.pallas import tpu as pltpu
from jax.experimental.pallas import tpu_sc as plsc

N_NODES = 50000
EMBED_DIM = 64
NUM_CORES = 2
NUM_SUBCORES = 16
NUM_WORKERS = NUM_CORES * NUM_SUBCORES
CHUNK = 80
NUM_CHUNKS = N_NODES // CHUNK
FULL_ROUNDS = NUM_CHUNKS // NUM_WORKERS
GRP = 16

_mesh = plsc.VectorSubcoreMesh(core_axis_name="c", subcore_axis_name="s")


@functools.partial(
    pl.kernel,
    mesh=_mesh,
    out_type=jax.ShapeDtypeStruct((N_NODES, EMBED_DIM + 1), jnp.float32),
    scratch_types=[
        pltpu.VMEM((CHUNK,), jnp.int32),
        pltpu.VMEM((CHUNK, 8, EMBED_DIM), jnp.float32),
        pltpu.VMEM((CHUNK, EMBED_DIM + 1), jnp.float32),
        pltpu.VMEM((CHUNK,), jnp.float32),
        pltpu.SemaphoreType.DMA,
    ],
)
def _gather_concat(table_hbm, idx_hbm, flag_hbm, out_hbm, idx_v, blocks_v,
                   out_v, flag_v, sem):
    wid = lax.axis_index("s") * NUM_CORES + lax.axis_index("c")
    last_lane = lax.iota(jnp.int32, 16) == 15

    def do_chunk(c):
        base = c * CHUNK
        pltpu.sync_copy(idx_hbm.at[pl.ds(base, CHUNK)], idx_v)
        pltpu.sync_copy(flag_hbm.at[pl.ds(base, CHUNK)], flag_v)

        def grp_body(g, carry):
            ivec = idx_v[pl.ds(g * GRP, GRP)]
            fvec = flag_v[pl.ds(g * GRP, GRP)]
            bvec = lax.bitwise_and(ivec, ~7)
            svec = lax.bitwise_and(ivec, 7)
            copies = []
            for t in range(GRP):
                r = g * GRP + t
                start = pl.multiple_of(bvec[t], 8)
                copies.append(pltpu.async_copy(
                    table_hbm.at[pl.ds(start, 8), :], blocks_v.at[r], sem))
            for t in range(GRP):
                copies[t].wait()
            for t in range(GRP):
                r = g * GRP + t
                for k in range(EMBED_DIM // 16):
                    out_v[r, pl.ds(k * 16, 16)] = (
                        blocks_v[r, svec[t], pl.ds(k * 16, 16)])
                cur = out_v[r, pl.ds(EMBED_DIM - 15, 16)]
                out_v[r, pl.ds(EMBED_DIM - 15, 16)] = jnp.where(
                    last_lane, lax.broadcast(fvec[t], (16,)), cur)
            return carry

        lax.fori_loop(0, CHUNK // GRP, grp_body, 0)
        pltpu.sync_copy(out_v, out_hbm.at[pl.ds(base, CHUNK)])

    for k in range(FULL_ROUNDS):
        do_chunk(wid + k * NUM_WORKERS)

    @pl.when(wid + FULL_ROUNDS * NUM_WORKERS < NUM_CHUNKS)
    def _():
        do_chunk(wid + FULL_ROUNDS * NUM_WORKERS)


def kernel(table, indices, is_candidate):
    return _gather_concat(table, indices.astype(jnp.int32), is_candidate)


# speedup vs baseline: 1.0909x; 1.0909x over previous
import functools

import jax
import jax.numpy as jnp
from jax import lax
from jax.experimental import pallas as pl
from jax.experimental.pallas import tpu as pltpu
from jax.experimental.pallas import tpu_sc as plsc

N_NODES = 50000
EMBED_DIM = 64
NUM_CORES = 2
NUM_SUBCORES = 16
NUM_WORKERS = NUM_CORES * NUM_SUBCORES
CHUNK = 80
NUM_CHUNKS = N_NODES // CHUNK
FULL_ROUNDS = NUM_CHUNKS // NUM_WORKERS
GRP = 16
NGRP = CHUNK // GRP

_mesh = plsc.VectorSubcoreMesh(core_axis_name="c", subcore_axis_name="s")


@functools.partial(
    pl.kernel,
    mesh=_mesh,
    out_type=jax.ShapeDtypeStruct((N_NODES, EMBED_DIM + 1), jnp.float32),
    scratch_types=[
        pltpu.VMEM((CHUNK,), jnp.int32),
        pltpu.VMEM((CHUNK, 8, EMBED_DIM), jnp.float32),
        pltpu.VMEM((CHUNK, EMBED_DIM + 1), jnp.float32),
        pltpu.VMEM((CHUNK,), jnp.float32),
        pltpu.SemaphoreType.DMA,
        pltpu.SemaphoreType.DMA,
    ],
)
def _gather_concat(table_hbm, idx_hbm, flag_hbm, out_hbm, idx_v, blocks_v,
                   out_v, flag_v, sem_a, sem_b):
    wid = lax.axis_index("s") * NUM_CORES + lax.axis_index("c")
    last_lane = lax.iota(jnp.int32, 16) == 15

    def issue_group(g):
        sem = sem_a if g % 2 == 0 else sem_b
        ivec = idx_v[pl.ds(g * GRP, GRP)]
        bvec = lax.bitwise_and(ivec, ~7)
        copies = []
        for t in range(GRP):
            start = pl.multiple_of(bvec[t], 8)
            copies.append(pltpu.async_copy(
                table_hbm.at[pl.ds(start, 8), :],
                blocks_v.at[g * GRP + t], sem))
        return copies

    def extract_group(g, copies):
        for c in copies:
            c.wait()
        ivec = idx_v[pl.ds(g * GRP, GRP)]
        fvec = flag_v[pl.ds(g * GRP, GRP)]
        svec = lax.bitwise_and(ivec, 7)
        for t in range(GRP):
            r = g * GRP + t
            for k in range(EMBED_DIM // 16):
                out_v[r, pl.ds(k * 16, 16)] = (
                    blocks_v[r, svec[t], pl.ds(k * 16, 16)])
            cur = out_v[r, pl.ds(EMBED_DIM - 15, 16)]
            out_v[r, pl.ds(EMBED_DIM - 15, 16)] = jnp.where(
                last_lane, lax.broadcast(fvec[t], (16,)), cur)

    def do_chunk(c):
        base = pl.multiple_of(c * CHUNK, 8)
        pltpu.sync_copy(idx_hbm.at[pl.ds(base, CHUNK)], idx_v)
        pltpu.sync_copy(flag_hbm.at[pl.ds(base, CHUNK)], flag_v)
        pending = issue_group(0)
        for g in range(NGRP):
            nxt = issue_group(g + 1) if g + 1 < NGRP else None
            extract_group(g, pending)
            pending = nxt
        pltpu.sync_copy(out_v, out_hbm.at[pl.ds(base, CHUNK)])

    def round_body(k, carry):
        do_chunk(wid + k * NUM_WORKERS)
        return carry

    lax.fori_loop(0, FULL_ROUNDS, round_body, 0)

    @pl.when(wid + FULL_ROUNDS * NUM_WORKERS < NUM_CHUNKS)
    def _():
        do_chunk(wid + FULL_ROUNDS * NUM_WORKERS)


def kernel(table, indices, is_candidate):
    return _gather_concat(table, indices.astype(jnp.int32), is_candidate)


# speedup vs baseline: 1.1063x; 1.0140x over previous
import functools

import jax
import jax.numpy as jnp
from jax import lax
from jax.experimental import pallas as pl
from jax.experimental.pallas import tpu as pltpu
from jax.experimental.pallas import tpu_sc as plsc

N_NODES = 50000
EMBED_DIM = 64
NUM_CORES = 2
NUM_SUBCORES = 16
NUM_WORKERS = NUM_CORES * NUM_SUBCORES
CHUNK = 80
NUM_CHUNKS = N_NODES // CHUNK
FULL_ROUNDS = NUM_CHUNKS // NUM_WORKERS
GRP = 16
NGRP = CHUNK // GRP

_mesh = plsc.VectorSubcoreMesh(core_axis_name="c", subcore_axis_name="s")


@functools.partial(
    pl.kernel,
    mesh=_mesh,
    out_type=jax.ShapeDtypeStruct((N_NODES, EMBED_DIM + 1), jnp.float32),
    scratch_types=[
        pltpu.VMEM((CHUNK,), jnp.int32),
        pltpu.VMEM((CHUNK, 8, EMBED_DIM), jnp.float32),
        pltpu.VMEM((CHUNK, EMBED_DIM + 1), jnp.float32),
        pltpu.VMEM((CHUNK,), jnp.float32),
        pltpu.SemaphoreType.DMA,
        pltpu.SemaphoreType.DMA,
        pltpu.SemaphoreType.DMA,
    ],
)
def _gather_concat(table_hbm, idx_hbm, flag_hbm, out_hbm, idx_v, blocks_v,
                   out_v, flag_v, sem_a, sem_b, sem_c):
    wid = lax.axis_index("s") * NUM_CORES + lax.axis_index("c")
    last_lane = lax.iota(jnp.int32, 16) == 15
    sems = (sem_a, sem_b, sem_c)

    def issue_group(g):
        sem = sems[g % 3]
        ivec = idx_v[pl.ds(g * GRP, GRP)]
        bvec = lax.bitwise_and(ivec, ~7)
        copies = []
        for t in range(GRP):
            start = pl.multiple_of(bvec[t], 8)
            copies.append(pltpu.async_copy(
                table_hbm.at[pl.ds(start, 8), :],
                blocks_v.at[g * GRP + t], sem))
        return copies

    def extract_group(g, copies):
        for c in copies:
            c.wait()
        ivec = idx_v[pl.ds(g * GRP, GRP)]
        fvec = flag_v[pl.ds(g * GRP, GRP)]
        svec = lax.bitwise_and(ivec, 7)
        for t in range(GRP):
            r = g * GRP + t
            for k in range(EMBED_DIM // 16):
                out_v[r, pl.ds(k * 16, 16)] = (
                    blocks_v[r, svec[t], pl.ds(k * 16, 16)])
            cur = out_v[r, pl.ds(EMBED_DIM - 15, 16)]
            out_v[r, pl.ds(EMBED_DIM - 15, 16)] = jnp.where(
                last_lane, lax.broadcast(fvec[t], (16,)), cur)

    def do_chunk(c):
        base = pl.multiple_of(c * CHUNK, 8)
        pltpu.sync_copy(idx_hbm.at[pl.ds(base, CHUNK)], idx_v)
        pltpu.sync_copy(flag_hbm.at[pl.ds(base, CHUNK)], flag_v)
        pending = {0: issue_group(0), 1: issue_group(1)}
        for g in range(NGRP):
            if g + 2 < NGRP:
                pending[g + 2] = issue_group(g + 2)
            extract_group(g, pending.pop(g))
        pltpu.sync_copy(out_v, out_hbm.at[pl.ds(base, CHUNK)])

    def round_body(k, carry):
        do_chunk(wid + k * NUM_WORKERS)
        return carry

    lax.fori_loop(0, FULL_ROUNDS, round_body, 0)

    @pl.when(wid + FULL_ROUNDS * NUM_WORKERS < NUM_CHUNKS)
    def _():
        do_chunk(wid + FULL_ROUNDS * NUM_WORKERS)


def kernel(table, indices, is_candidate):
    return _gather_concat(table, indices.astype(jnp.int32), is_candidate)


# speedup vs baseline: 1.1590x; 1.0477x over previous
import functools

import jax
import jax.numpy as jnp
from jax import lax
from jax.experimental import pallas as pl
from jax.experimental.pallas import tpu as pltpu
from jax.experimental.pallas import tpu_sc as plsc

N_NODES = 50000
EMBED_DIM = 64
NUM_CORES = 2
NUM_SUBCORES = 16
NUM_WORKERS = NUM_CORES * NUM_SUBCORES
WRANGE = 1600
WRANGE_LAST = N_NODES - WRANGE * (NUM_WORKERS - 1)
CHUNK = 80
GRP = 16
NGRP = CHUNK // GRP

_mesh = plsc.VectorSubcoreMesh(core_axis_name="c", subcore_axis_name="s")


@functools.partial(
    pl.kernel,
    mesh=_mesh,
    out_type=jax.ShapeDtypeStruct((N_NODES, EMBED_DIM + 1), jnp.float32),
    scratch_types=[
        pltpu.VMEM((WRANGE,), jnp.int32),
        pltpu.VMEM((CHUNK, 8, EMBED_DIM), jnp.float32),
        pltpu.VMEM((CHUNK, EMBED_DIM + 1), jnp.float32),
        pltpu.VMEM((WRANGE,), jnp.float32),
        pltpu.SemaphoreType.DMA,
        pltpu.SemaphoreType.DMA,
        pltpu.SemaphoreType.DMA,
    ],
)
def _gather_concat(table_hbm, idx_hbm, flag_hbm, out_hbm, idx_v, blocks_v,
                   out_v, flag_v, sem_a, sem_b, sem_c):
    wid = lax.axis_index("s") * NUM_CORES + lax.axis_index("c")
    last_lane = lax.iota(jnp.int32, 16) == 15
    sems = (sem_a, sem_b, sem_c)
    wbase = pl.multiple_of(wid * WRANGE, 8)
    is_last = wid == NUM_WORKERS - 1

    @pl.when(jnp.logical_not(is_last))
    def _():
        pltpu.sync_copy(idx_hbm.at[pl.ds(wbase, WRANGE)], idx_v)
        pltpu.sync_copy(flag_hbm.at[pl.ds(wbase, WRANGE)], flag_v)

    @pl.when(is_last)
    def _():
        pltpu.sync_copy(idx_hbm.at[pl.ds(wbase, WRANGE_LAST)],
                        idx_v.at[pl.ds(0, WRANGE_LAST)])
        pltpu.sync_copy(flag_hbm.at[pl.ds(wbase, WRANGE_LAST)],
                        flag_v.at[pl.ds(0, WRANGE_LAST)])

    def issue_group(off, g):
        sem = sems[g % 3]
        ivec = idx_v[pl.ds(off + g * GRP, GRP)]
        bvec = lax.bitwise_and(ivec, ~7)
        copies = []
        for t in range(GRP):
            start = pl.multiple_of(bvec[t], 8)
            copies.append(pltpu.async_copy(
                table_hbm.at[pl.ds(start, 8), :],
                blocks_v.at[g * GRP + t], sem))
        return copies

    def extract_group(off, g, copies):
        for c in copies:
            c.wait()
        ivec = idx_v[pl.ds(off + g * GRP, GRP)]
        fvec = flag_v[pl.ds(off + g * GRP, GRP)]
        svec = lax.bitwise_and(ivec, 7)
        for t in range(GRP):
            r = g * GRP + t
            for k in range(EMBED_DIM // 16):
                out_v[r, pl.ds(k * 16, 16)] = (
                    blocks_v[r, svec[t], pl.ds(k * 16, 16)])
            cur = out_v[r, pl.ds(EMBED_DIM - 15, 16)]
            out_v[r, pl.ds(EMBED_DIM - 15, 16)] = jnp.where(
                last_lane, lax.broadcast(fvec[t], (16,)), cur)

    def chunk_body(j, carry):
        off = j * CHUNK
        pending = {0: issue_group(off, 0), 1: issue_group(off, 1)}
        for g in range(NGRP):
            if g + 2 < NGRP:
                pending[g + 2] = issue_group(off, g + 2)
            extract_group(off, g, pending.pop(g))
        obase = pl.multiple_of(wbase + off, 8)
        pltpu.sync_copy(out_v, out_hbm.at[pl.ds(obase, CHUNK)])
        return carry

    n_chunks = jnp.where(is_last, WRANGE_LAST // CHUNK, WRANGE // CHUNK)
    lax.fori_loop(0, n_chunks, chunk_body, 0)


def kernel(table, indices, is_candidate):
    return _gather_concat(table, indices.astype(jnp.int32), is_candidate)


# speedup vs baseline: 1.1766x; 1.0152x over previous
import functools

import jax
import jax.numpy as jnp
from jax import lax
from jax.experimental import pallas as pl
from jax.experimental.pallas import tpu as pltpu
from jax.experimental.pallas import tpu_sc as plsc

N_NODES = 50000
EMBED_DIM = 64
NUM_CORES = 2
NUM_SUBCORES = 16
NUM_WORKERS = NUM_CORES * NUM_SUBCORES
WRANGE = 1600
WRANGE_LAST = N_NODES - WRANGE * (NUM_WORKERS - 1)
CHUNK = 80
GRP = 16
NGRP = CHUNK // GRP

_mesh = plsc.VectorSubcoreMesh(core_axis_name="c", subcore_axis_name="s")


@functools.partial(
    pl.kernel,
    mesh=_mesh,
    out_type=jax.ShapeDtypeStruct((N_NODES, EMBED_DIM + 1), jnp.float32),
    scratch_types=[
        pltpu.VMEM((WRANGE,), jnp.int32),
        pltpu.VMEM((CHUNK, 8, EMBED_DIM), jnp.float32),
        pltpu.VMEM((CHUNK, EMBED_DIM + 1), jnp.float32),
        pltpu.VMEM((WRANGE,), jnp.float32),
        pltpu.SemaphoreType.DMA,
        pltpu.SemaphoreType.DMA,
        pltpu.SemaphoreType.DMA,
        pltpu.SemaphoreType.DMA,
    ],
)
def _gather_concat(table_hbm, idx_hbm, flag_hbm, out_hbm, idx_v, blocks_v,
                   out_v, flag_v, sem_a, sem_b, sem_c, sem_out):
    wid = lax.axis_index("s") * NUM_CORES + lax.axis_index("c")
    last_lane = lax.iota(jnp.int32, 16) == 15
    sems = (sem_a, sem_b, sem_c)
    wbase = pl.multiple_of(wid * WRANGE, 8)
    is_last = wid == NUM_WORKERS - 1

    @pl.when(jnp.logical_not(is_last))
    def _():
        pltpu.sync_copy(idx_hbm.at[pl.ds(wbase, WRANGE)], idx_v)
        pltpu.sync_copy(flag_hbm.at[pl.ds(wbase, WRANGE)], flag_v)

    @pl.when(is_last)
    def _():
        pltpu.sync_copy(idx_hbm.at[pl.ds(wbase, WRANGE_LAST)],
                        idx_v.at[pl.ds(0, WRANGE_LAST)])
        pltpu.sync_copy(flag_hbm.at[pl.ds(wbase, WRANGE_LAST)],
                        flag_v.at[pl.ds(0, WRANGE_LAST)])

    def issue_group(off, g):
        sem = sems[g % 3]
        ivec = idx_v[pl.ds(off + g * GRP, GRP)]
        bvec = lax.bitwise_and(ivec, ~7)
        copies = []
        for t in range(GRP):
            start = pl.multiple_of(bvec[t], 8)
            copies.append(pltpu.async_copy(
                table_hbm.at[pl.ds(start, 8), :],
                blocks_v.at[g * GRP + t], sem))
        return copies

    def extract_group(off, g, copies):
        for c in copies:
            c.wait()
        ivec = idx_v[pl.ds(off + g * GRP, GRP)]
        fvec = flag_v[pl.ds(off + g * GRP, GRP)]
        svec = lax.bitwise_and(ivec, 7)
        for t in range(GRP):
            r = g * GRP + t
            for k in range(EMBED_DIM // 16):
                out_v[r, pl.ds(k * 16, 16)] = (
                    blocks_v[r, svec[t], pl.ds(k * 16, 16)])
            cur = out_v[r, pl.ds(EMBED_DIM - 15, 16)]
            out_v[r, pl.ds(EMBED_DIM - 15, 16)] = jnp.where(
                last_lane, lax.broadcast(fvec[t], (16,)), cur)

    def chunk_body(j, carry):
        off = j * CHUNK
        obase = pl.multiple_of(wbase + off, 8)
        pending = {0: issue_group(off, 0), 1: issue_group(off, 1)}

        @pl.when(j > 0)
        def _():
            pltpu.make_async_copy(
                out_v, out_hbm.at[pl.ds(obase, CHUNK)], sem_out).wait()

        for g in range(NGRP):
            if g + 2 < NGRP:
                pending[g + 2] = issue_group(off, g + 2)
            extract_group(off, g, pending.pop(g))
        pltpu.async_copy(out_v, out_hbm.at[pl.ds(obase, CHUNK)], sem_out)
        return carry

    n_chunks = jnp.where(is_last, WRANGE_LAST // CHUNK, WRANGE // CHUNK)
    lax.fori_loop(0, n_chunks, chunk_body, 0)
    pltpu.make_async_copy(
        out_v, out_hbm.at[pl.ds(wbase, CHUNK)], sem_out).wait()


def kernel(table, indices, is_candidate):
    return _gather_concat(table, indices.astype(jnp.int32), is_candidate)
